# baseline (speedup 1.0000x reference)
import functools

import jax
import jax.numpy as jnp
from jax import lax
from jax.experimental import pallas as pl
from jax.experimental.pallas import tpu as pltpu
from jax.experimental.pallas import tpu_sc as plsc

_VOCAB = 100000
_D = 32
_NTOK = 512

_NC, _NS = 2, 16
_NW = _NC * _NS
_TOK_PER_W = _NTOK // _NW


def _build_sc_gather():
    mesh = plsc.VectorSubcoreMesh(core_axis_name="c", subcore_axis_name="s")

    @functools.partial(
        pl.kernel,
        mesh=mesh,
        compiler_params=pltpu.CompilerParams(needs_layout_passes=False),
        out_type=jax.ShapeDtypeStruct((_NTOK, _D), jnp.float32),
        scratch_types=[
            pltpu.VMEM((_TOK_PER_W,), jnp.int32),
            pltpu.VMEM((_TOK_PER_W, _D), jnp.float32),
            pltpu.SemaphoreType.DMA,
        ],
    )
    def sc_gather(table_hbm, idx_hbm, out_hbm, idx_v, rows_v, sem):
        wid = lax.axis_index("s") * _NC + lax.axis_index("c")
        base = wid * _TOK_PER_W
        pltpu.sync_copy(idx_hbm.at[pl.ds(base, _TOK_PER_W)], idx_v)
        ivec = idx_v[...]
        copies = []
        for t in range(_TOK_PER_W):
            copies.append(
                pltpu.make_async_copy(
                    table_hbm.at[pl.ds(ivec[t], 1)],
                    rows_v.at[pl.ds(t, 1)],
                    sem,
                )
            )
            copies[-1].start()
        for c in copies:
            c.wait()
        pltpu.sync_copy(rows_v, out_hbm.at[pl.ds(base, _TOK_PER_W)])

    return sc_gather


_TM = 32


def _matmul_body(emb_ref, w_ref, b_ref, out_ref):
    out_ref[...] = (
        jnp.dot(emb_ref[...], w_ref[...], preferred_element_type=jnp.float32)
        + b_ref[...]
    )


@jax.jit
def kernel(x, table, W, b):
    B, T = x.shape
    idx = x.reshape(_NTOK)
    tok_emb = jnp.take(table, idx, axis=0)

    nm = _NTOK // _TM
    logits = pl.pallas_call(
        _matmul_body,
        grid=(nm,),
        in_specs=[
            pl.BlockSpec((_TM, _D), lambda i: (i, 0)),
            pl.BlockSpec((_D, _VOCAB), lambda i: (0, 0)),
            pl.BlockSpec((1, _VOCAB), lambda i: (0, 0)),
        ],
        out_specs=pl.BlockSpec((_TM, _VOCAB), lambda i: (i, 0)),
        out_shape=jax.ShapeDtypeStruct((_NTOK, _VOCAB), jnp.float32),
        compiler_params=pltpu.CompilerParams(
            vmem_limit_bytes=128 * 1024 * 1024,
            dimension_semantics=("parallel",),
        ),
    )(tok_emb, W, b.reshape(1, _VOCAB))
    return logits.reshape(B, T, _VOCAB)
